# baseline (speedup 1.0000x reference)
import functools

import numpy as np
import jax
import jax.numpy as jnp
from jax import lax
from jax.experimental import pallas as pl
from jax.experimental.pallas import tpu as pltpu
from jax.experimental.pallas import tpu_sc as plsc

MUL = 16
F = 4 * MUL
SQRT2 = 2.0 ** 0.5
SQRT3 = 3.0 ** 0.5
SQRT5 = 5.0 ** 0.5

NC, NS = 2, 16
NW = NC * NS
K = 128
G = 5
GK = G * K
GS = 2
GKS = GS * K
HALF = F // 2

NB_NODE = 2000
EB_EDGE = 3200


def _permute_body(x_ref, p_ref, o_ref):
    o_ref[...] = jnp.dot(x_ref[...], p_ref[...],
                         preferred_element_type=jnp.float32)


def _final_body(a_ref, wb_ref, nf_ref, o_ref):
    wb = wb_ref[...]
    lin = jnp.dot(a_ref[0], wb[:HALF, :], preferred_element_type=jnp.float32)
    lin += jnp.dot(a_ref[1], wb[HALF:, :], preferred_element_type=jnp.float32)
    o_ref[...] = lin + nf_ref[...]


def _tdot(lhs_t, rhs):
    return lax.dot_general(lhs_t, rhs, (((0,), (0,)), ((), ())),
                           preferred_element_type=jnp.float32)


def _msg_body(evt_ref, x_ref, w1_ref, w2_ref, wq3_ref, xm_ref, gm_ref,
              cc_ref, cm_ref, o_ref):
    evt = evt_ref[...]
    sq = evt * evt
    r2 = sq[0:1, :] + sq[1:2, :] + sq[2:3, :] + 1e-12
    rr = jnp.sqrt(r2)
    inv = 1.0 / rr
    inv2 = inv * inv
    n_t = evt * inv
    na_t = sq * inv2
    nrot_t = jnp.concatenate([n_t[1:3, :], n_t[0:1, :]], axis=0)
    nall_t = jnp.concatenate([n_t, na_t, n_t * nrot_t], axis=0)
    h = _tdot(rr, w1_ref[...])
    h = h * jax.nn.sigmoid(h)
    h = jnp.dot(h, w2_ref[...], preferred_element_type=jnp.float32)
    h = h * jax.nn.sigmoid(h)
    wq = jnp.dot(h, wq3_ref[...], preferred_element_type=jnp.float32)
    xcat = jnp.dot(x_ref[...], xm_ref[...], preferred_element_type=jnp.float32)
    gcat = _tdot(nall_t, gm_ref[...]) + cc_ref[...]
    z = (xcat * gcat * wq).astype(jnp.bfloat16)
    o_ref[...] = jnp.dot(z, cm_ref[...], preferred_element_type=jnp.float32)


SQRT15 = 15.0 ** 0.5


def _msg_consts6():
    u = np.arange(MUL)
    w6 = 6 * F

    def sub(b, c):
        return F * b + MUL * c + u

    def blk(b):
        return MUL * b + u

    qm = np.zeros((80, w6), np.float32)
    qm[u, sub(0, 0)] = 1.0 / SQRT2
    for c in (1, 2, 3):
        qm[16 + u, sub(0, c)] = 1.0 / SQRT3
        qm[64 + u, sub(1, c)] = 1.0 / SQRT15
        qm[64 + u, sub(2, c)] = 1.0 / SQRT15
        qm[64 + u, sub(3, c)] = 1.0 / SQRT15
        qm[48 + u, sub(4, c)] = 1.0 / (SQRT3 * SQRT2)
        qm[32 + u, sub(5, c)] = 1.0 / SQRT3
    s0 = np.zeros((F, F), np.float32)
    for b in range(4):
        s0[u, blk(b)] = 1.0
    mr1 = np.zeros((F, F), np.float32)
    mr1[blk(2), blk(1)] = 1.0
    mr1[blk(3), blk(2)] = 1.0
    mr1[blk(1), blk(3)] = 1.0
    mr2 = np.zeros((F, F), np.float32)
    mr2[blk(3), blk(1)] = 1.0
    mr2[blk(1), blk(2)] = 1.0
    mr2[blk(2), blk(3)] = 1.0
    eye = np.eye(F, dtype=np.float32)
    xm = np.concatenate([s0, eye, mr1, mr2, eye, eye], axis=1)
    xm = _perm_matrix() @ xm
    gm = np.zeros((9, w6), np.float32)
    for c in (1, 2, 3):
        gm[c - 1, sub(0, c)] = SQRT3
        gm[c - 1, sub(4, c)] = SQRT3
    gm[3, sub(1, 1)] = SQRT15 / 2
    gm[4, sub(1, 1)] = -SQRT15 / 2
    gm[5, sub(1, 1)] = -SQRT15 / 2
    gm[3, sub(1, 2)] = -SQRT15 / 2
    gm[4, sub(1, 2)] = SQRT15 / 2
    gm[5, sub(1, 2)] = -SQRT15 / 2
    gm[5, sub(1, 3)] = SQRT15
    gm[6, sub(2, 1)] = SQRT15
    gm[7, sub(2, 2)] = SQRT15
    gm[8, sub(2, 3)] = SQRT15
    gm[8, sub(3, 1)] = SQRT15
    gm[6, sub(3, 2)] = SQRT15
    gm[7, sub(3, 3)] = SQRT15
    cc = np.zeros((1, w6), np.float32)
    cc[0, sub(0, 0)] = 1.0
    cc[0, sub(1, 1)] = SQRT15 / 6
    cc[0, sub(1, 2)] = SQRT15 / 6
    cc[0, sub(1, 3)] = -SQRT15 / 3
    cc[0, F * 5:F * 6] = 1.0
    cm = np.zeros((w6, F), np.float32)
    for b in (0, 1, 2, 3, 5):
        for c in range(4):
            cm[sub(b, c), MUL * c + u] = 1.0
    for c in (1, 2, 3):
        cm[sub(4, c), u] = 1.0
    return qm, xm, gm, cc, cm


def _run_msg(edge_vec, x_src, W1, W2, W3, interpret=False):
    n_edges = x_src.shape[0]
    qm, xm, gm, cc, cm = _msg_consts6()
    wq3 = jnp.dot(W3 * 0.125, jnp.asarray(qm))
    xm16 = jnp.asarray(xm, dtype=jnp.bfloat16)
    cm16 = jnp.asarray(cm, dtype=jnp.bfloat16)
    gmj, ccj = jnp.asarray(gm), jnp.asarray(cc)
    consts = [wq3, xm16, gmj, ccj, cm16]
    cspecs = [pl.BlockSpec(c.shape, lambda i: (0, 0)) for c in consts]
    return pl.pallas_call(
        _msg_body,
        grid=(n_edges // EB_EDGE,),
        in_specs=[pl.BlockSpec((3, EB_EDGE), lambda i: (0, i)),
                  pl.BlockSpec((EB_EDGE, F), lambda i: (i, 0)),
                  pl.BlockSpec(W1.shape, lambda i: (0, 0)),
                  pl.BlockSpec(W2.shape, lambda i: (0, 0))] + cspecs,
        out_specs=pl.BlockSpec((EB_EDGE, F), lambda i: (i, 0)),
        out_shape=jax.ShapeDtypeStruct((n_edges, F), jnp.float32),
        interpret=interpret,
    )(edge_vec.T, x_src, W1, W2 * 0.125, *consts)


def _perm_matrix():
    p = np.zeros((F, F), np.float32)
    p[np.arange(MUL), np.arange(MUL)] = 1.0
    for c in range(3):
        for u in range(MUL):
            p[MUL + 3 * u + c, MUL + MUL * c + u] = 1.0
    return p


def _folded_linear(wl0, wl1):
    wb = jnp.zeros((F, F), jnp.float32)
    wb = wb.at[:MUL, :MUL].set(wl0 * 0.25)
    cc, uu, vv = np.meshgrid(np.arange(3), np.arange(MUL), np.arange(MUL),
                             indexing="ij")
    rows = MUL + MUL * cc + uu
    cols = MUL + 3 * vv + cc
    vals = jnp.broadcast_to(wl1 * 0.25, (3, MUL, MUL))
    return wb.at[rows, cols].set(vals)


def kernel(node_feat, edge_index, edge_vec, W1, W2, W3, Wl0, Wl1):
    n_nodes = node_feat.shape[0]
    n_edges = edge_vec.shape[0]
    assert n_edges % (K * G) == 0 and n_edges % (K * GS) == 0
    nch = n_edges // K
    ng = nch // G
    ngs = nch // GS
    pw_g = (-(-ng // NW) + 1) // 2 * 2
    pt_g = (-(-ngs // NS) + 1) // 2 * 2
    n_pad = -(-n_nodes // (NS * 8)) * NS * 8
    rows_t = n_pad // NS

    src_r = edge_index[0].reshape(nch, K)
    dst_r = edge_index[1].reshape(nch, K)

    mesh = plsc.VectorSubcoreMesh(core_axis_name="c", subcore_axis_name="s",
                                  num_cores=NC, num_subcores=NS)
    scp = pltpu.CompilerParams(use_tc_tiling_on_sc=False)

    node_bf = node_feat.astype(jnp.bfloat16)

    @functools.partial(
        pl.kernel,
        out_type=jax.ShapeDtypeStruct((n_edges, F), jnp.bfloat16),
        mesh=mesh,
        scratch_types=[pltpu.VMEM((2, G, K), jnp.int32),
                       pltpu.VMEM((2, GK, F), jnp.bfloat16),
                       pltpu.SemaphoreType.DMA((2,)),
                       pltpu.SemaphoreType.DMA((2,)),
                       pltpu.SemaphoreType.DMA((2,))],
        compiler_params=scp,
    )
    def _gather(nf_hbm, srcr_hbm, x_hbm, idxb, rowsb, isem, gsem, wsem):
        wid = lax.axis_index("s") * NC + lax.axis_index("c")

        def idx_copy(g, b):
            return pltpu.make_async_copy(srcr_hbm.at[pl.ds(g * G, G)],
                                         idxb.at[b], isem.at[b])

        def row_write(g, b):
            return pltpu.make_async_copy(rowsb.at[b],
                                         x_hbm.at[pl.ds(g * GK, GK)],
                                         wsem.at[b])

        def gather_drain(b):
            return pltpu.make_async_copy(nf_hbm.at[pl.ds(0, GK)],
                                         rowsb.at[b], gsem.at[b])

        @pl.when(wid < ng)
        def _():
            idx_copy(wid, 0).start()

        def body(q, carry):
            for b in (0, 1):
                gi = q * 2 + b
                g = wid + gi * NW

                @pl.when(g + NW < ng)
                def _():
                    idx_copy(g + NW, 1 - b).start()

                @pl.when((gi >= 2) & (g - 2 * NW < ng))
                def _():
                    row_write(g - 2 * NW, b).wait()

                @pl.when(g < ng)
                def _():
                    idx_copy(g, b).wait()
                    for j in range(G):
                        pltpu.async_copy(nf_hbm.at[idxb.at[b, j]],
                                         rowsb.at[b, pl.ds(j * K, K)],
                                         gsem.at[b])
                    gather_drain(b).wait()
                    row_write(g, b).start()
            return carry

        lax.fori_loop(0, pw_g // 2, body, 0)
        for t in (pw_g - 2, pw_g - 1):
            g = wid + t * NW

            @pl.when(g < ng)
            def _():
                row_write(g, t % 2).wait()

    x_src = _gather(node_bf, src_r)

    msg = _run_msg(edge_vec, x_src, W1, W2, W3)
    msg = _run_msg(edge_vec, x_src, W1, W2, W3)

    zinit = jnp.zeros((n_pad, HALF), jnp.float32)

    @functools.partial(
        pl.kernel,
        out_type=jax.ShapeDtypeStruct((NC, n_pad, HALF), jnp.float32),
        mesh=mesh,
        scratch_types=[pltpu.VMEM((2, GS, K), jnp.int32),
                       pltpu.VMEM((2, GKS, HALF), jnp.float32),
                       pltpu.VMEM_SHARED((n_pad, HALF), jnp.float32),
                       pltpu.SemaphoreType.DMA((2,)),
                       pltpu.SemaphoreType.DMA((2,)),
                       pltpu.SemaphoreType.DMA((2,))],
        compiler_params=scp,
    )
    def _scatter(dstr_hbm, msg_hbm, z_hbm, out_hbm, didxb, mb, acc_sh,
                 isem, msem, ssem):
        cid = lax.axis_index("c")
        sid = lax.axis_index("s")
        row0 = sid * rows_t
        pltpu.sync_copy(z_hbm.at[pl.ds(row0, rows_t)],
                        acc_sh.at[pl.ds(row0, rows_t)])
        plsc.subcore_barrier()

        def idx_copy(g, b):
            return pltpu.make_async_copy(dstr_hbm.at[pl.ds(g * GS, GS)],
                                         didxb.at[b], isem.at[b])

        def msg_copy(g, b):
            return pltpu.make_async_copy(
                msg_hbm.at[pl.ds(g * GKS, GKS), pl.ds(cid * HALF, HALF)],
                mb.at[b], msem.at[b])

        def scat_drain(b):
            return pltpu.make_async_copy(mb.at[b], acc_sh.at[pl.ds(0, GKS)],
                                         ssem.at[b])

        @pl.when(sid < ngs)
        def _():
            idx_copy(sid, 0).start()
            msg_copy(sid, 0).start()

        def body(q, carry):
            for b in (0, 1):
                gi = q * 2 + b
                g = sid + gi * NS

                @pl.when((gi >= 1) & (g - NS < ngs))
                def _():
                    scat_drain(1 - b).wait()

                @pl.when(g + NS < ngs)
                def _():
                    idx_copy(g + NS, 1 - b).start()
                    msg_copy(g + NS, 1 - b).start()

                @pl.when(g < ngs)
                def _():
                    idx_copy(g, b).wait()
                    msg_copy(g, b).wait()
                    for j in range(GS):
                        pltpu.async_copy(mb.at[b, pl.ds(j * K, K)],
                                         acc_sh.at[didxb.at[b, j]],
                                         ssem.at[b], add=True)
            return carry

        lax.fori_loop(0, pt_g // 2, body, 0)
        t = pt_g - 1
        g_last = sid + t * NS

        @pl.when(g_last < ngs)
        def _():
            scat_drain(t % 2).wait()
        plsc.subcore_barrier()
        pltpu.sync_copy(acc_sh.at[pl.ds(row0, rows_t)],
                        out_hbm.at[cid, pl.ds(row0, rows_t)])

    aggr2 = _scatter(dst_r, msg, zinit)

    wbig = _folded_linear(Wl0, Wl1)
    out = pl.pallas_call(
        _final_body,
        grid=(n_nodes // NB_NODE,),
        in_specs=[pl.BlockSpec((NC, NB_NODE, HALF), lambda i: (0, i, 0)),
                  pl.BlockSpec((F, F), lambda i: (0, 0)),
                  pl.BlockSpec((NB_NODE, F), lambda i: (i, 0))],
        out_specs=pl.BlockSpec((NB_NODE, F), lambda i: (i, 0)),
        out_shape=jax.ShapeDtypeStruct((n_nodes, F), jnp.float32),
    )(aggr2, wbig, node_feat)
    return out


# speedup vs baseline: 1.1084x; 1.1084x over previous
import functools

import numpy as np
import jax
import jax.numpy as jnp
from jax import lax
from jax.experimental import pallas as pl
from jax.experimental.pallas import tpu as pltpu
from jax.experimental.pallas import tpu_sc as plsc

MUL = 16
F = 4 * MUL
SQRT2 = 2.0 ** 0.5
SQRT3 = 3.0 ** 0.5
SQRT5 = 5.0 ** 0.5

NC, NS = 2, 16
NW = NC * NS
K = 128
G = 5
GK = G * K
GS = 2
GKS = GS * K
HALF = F // 2

NB_NODE = 2000
EB_EDGE = 6400


def _permute_body(x_ref, p_ref, o_ref):
    o_ref[...] = jnp.dot(x_ref[...], p_ref[...],
                         preferred_element_type=jnp.float32)


def _final_body(a_ref, wb_ref, nf_ref, o_ref):
    wb = wb_ref[...]
    lin = jnp.dot(a_ref[0], wb[:HALF, :], preferred_element_type=jnp.float32)
    lin += jnp.dot(a_ref[1], wb[HALF:, :], preferred_element_type=jnp.float32)
    o_ref[...] = lin + nf_ref[...]


def _tdot(lhs_t, rhs):
    return lax.dot_general(lhs_t, rhs, (((0,), (0,)), ((), ())),
                           preferred_element_type=jnp.float32)


def _msg_body(evt_ref, x_ref, w1_ref, w2_ref, wq3_ref, xm_ref, gm_ref,
              cc_ref, cm_ref, o_ref):
    evt = evt_ref[...]
    sq = evt * evt
    r2 = sq[0:1, :] + sq[1:2, :] + sq[2:3, :] + 1e-12
    rr = jnp.sqrt(r2)
    inv = 1.0 / rr
    inv2 = inv * inv
    n_t = evt * inv
    na_t = sq * inv2
    nrot_t = jnp.concatenate([n_t[1:3, :], n_t[0:1, :]], axis=0)
    nall_t = jnp.concatenate([n_t, na_t, n_t * nrot_t], axis=0)
    h = _tdot(rr, w1_ref[...])
    h = h * jax.nn.sigmoid(h)
    h = jnp.dot(h, w2_ref[...], preferred_element_type=jnp.float32)
    h = h * jax.nn.sigmoid(h)
    wq = jnp.dot(h, wq3_ref[...], preferred_element_type=jnp.float32)
    xcat = jnp.dot(x_ref[...], xm_ref[...], preferred_element_type=jnp.float32)
    gcat = _tdot(nall_t, gm_ref[...]) + cc_ref[...]
    z = (xcat * gcat * wq).astype(jnp.bfloat16)
    o_ref[...] = jnp.dot(z, cm_ref[...], preferred_element_type=jnp.float32)


SQRT15 = 15.0 ** 0.5


def _msg_consts6():
    u = np.arange(MUL)
    w6 = 6 * F

    def sub(b, c):
        return F * b + MUL * c + u

    def blk(b):
        return MUL * b + u

    qm = np.zeros((80, w6), np.float32)
    qm[u, sub(0, 0)] = 1.0 / SQRT2
    for c in (1, 2, 3):
        qm[16 + u, sub(0, c)] = 1.0 / SQRT3
        qm[64 + u, sub(1, c)] = 1.0 / SQRT15
        qm[64 + u, sub(2, c)] = 1.0 / SQRT15
        qm[64 + u, sub(3, c)] = 1.0 / SQRT15
        qm[48 + u, sub(4, c)] = 1.0 / (SQRT3 * SQRT2)
        qm[32 + u, sub(5, c)] = 1.0 / SQRT3
    s0 = np.zeros((F, F), np.float32)
    for b in range(4):
        s0[u, blk(b)] = 1.0
    mr1 = np.zeros((F, F), np.float32)
    mr1[blk(2), blk(1)] = 1.0
    mr1[blk(3), blk(2)] = 1.0
    mr1[blk(1), blk(3)] = 1.0
    mr2 = np.zeros((F, F), np.float32)
    mr2[blk(3), blk(1)] = 1.0
    mr2[blk(1), blk(2)] = 1.0
    mr2[blk(2), blk(3)] = 1.0
    eye = np.eye(F, dtype=np.float32)
    xm = np.concatenate([s0, eye, mr1, mr2, eye, eye], axis=1)
    xm = _perm_matrix() @ xm
    gm = np.zeros((9, w6), np.float32)
    for c in (1, 2, 3):
        gm[c - 1, sub(0, c)] = SQRT3
        gm[c - 1, sub(4, c)] = SQRT3
    gm[3, sub(1, 1)] = SQRT15 / 2
    gm[4, sub(1, 1)] = -SQRT15 / 2
    gm[5, sub(1, 1)] = -SQRT15 / 2
    gm[3, sub(1, 2)] = -SQRT15 / 2
    gm[4, sub(1, 2)] = SQRT15 / 2
    gm[5, sub(1, 2)] = -SQRT15 / 2
    gm[5, sub(1, 3)] = SQRT15
    gm[6, sub(2, 1)] = SQRT15
    gm[7, sub(2, 2)] = SQRT15
    gm[8, sub(2, 3)] = SQRT15
    gm[8, sub(3, 1)] = SQRT15
    gm[6, sub(3, 2)] = SQRT15
    gm[7, sub(3, 3)] = SQRT15
    cc = np.zeros((1, w6), np.float32)
    cc[0, sub(0, 0)] = 1.0
    cc[0, sub(1, 1)] = SQRT15 / 6
    cc[0, sub(1, 2)] = SQRT15 / 6
    cc[0, sub(1, 3)] = -SQRT15 / 3
    cc[0, F * 5:F * 6] = 1.0
    cm = np.zeros((w6, F), np.float32)
    for b in (0, 1, 2, 3, 5):
        for c in range(4):
            cm[sub(b, c), MUL * c + u] = 1.0
    for c in (1, 2, 3):
        cm[sub(4, c), u] = 1.0
    return qm, xm, gm, cc, cm


def _run_msg(edge_vec, x_src, W1, W2, W3, interpret=False):
    n_edges = x_src.shape[0]
    qm, xm, gm, cc, cm = _msg_consts6()
    wq3 = jnp.dot(W3 * 0.125, jnp.asarray(qm))
    xmj = jnp.asarray(xm)
    cm16 = jnp.asarray(cm, dtype=jnp.bfloat16)
    gmj, ccj = jnp.asarray(gm), jnp.asarray(cc)
    consts = [wq3, xmj, gmj, ccj, cm16]
    cspecs = [pl.BlockSpec(c.shape, lambda i: (0, 0)) for c in consts]
    return pl.pallas_call(
        _msg_body,
        grid=(n_edges // EB_EDGE,),
        in_specs=[pl.BlockSpec((3, EB_EDGE), lambda i: (0, i)),
                  pl.BlockSpec((EB_EDGE, F), lambda i: (i, 0)),
                  pl.BlockSpec(W1.shape, lambda i: (0, 0)),
                  pl.BlockSpec(W2.shape, lambda i: (0, 0))] + cspecs,
        out_specs=pl.BlockSpec((EB_EDGE, F), lambda i: (i, 0)),
        out_shape=jax.ShapeDtypeStruct((n_edges, F), jnp.float32),
        interpret=interpret,
    )(edge_vec.T, x_src, W1, W2 * 0.125, *consts)


def _perm_matrix():
    p = np.zeros((F, F), np.float32)
    p[np.arange(MUL), np.arange(MUL)] = 1.0
    for c in range(3):
        for u in range(MUL):
            p[MUL + 3 * u + c, MUL + MUL * c + u] = 1.0
    return p


def _folded_linear(wl0, wl1):
    wb = jnp.zeros((F, F), jnp.float32)
    wb = wb.at[:MUL, :MUL].set(wl0 * 0.25)
    cc, uu, vv = np.meshgrid(np.arange(3), np.arange(MUL), np.arange(MUL),
                             indexing="ij")
    rows = MUL + MUL * cc + uu
    cols = MUL + 3 * vv + cc
    vals = jnp.broadcast_to(wl1 * 0.25, (3, MUL, MUL))
    return wb.at[rows, cols].set(vals)


def kernel(node_feat, edge_index, edge_vec, W1, W2, W3, Wl0, Wl1):
    n_nodes = node_feat.shape[0]
    n_edges = edge_vec.shape[0]
    assert n_edges % (K * G) == 0 and n_edges % (K * GS) == 0
    nch = n_edges // K
    ng = nch // G
    ngs = nch // GS
    pw_g = (-(-ng // NW) + 1) // 2 * 2
    pt_g = (-(-ngs // NS) + 1) // 2 * 2
    n_pad = -(-n_nodes // (NS * 8)) * NS * 8
    rows_t = n_pad // NS

    src_r = edge_index[0].reshape(nch, K)
    dst_r = edge_index[1].reshape(nch, K)

    mesh = plsc.VectorSubcoreMesh(core_axis_name="c", subcore_axis_name="s",
                                  num_cores=NC, num_subcores=NS)
    scp = pltpu.CompilerParams(use_tc_tiling_on_sc=False)

    @functools.partial(
        pl.kernel,
        out_type=jax.ShapeDtypeStruct((n_edges, F), jnp.float32),
        mesh=mesh,
        scratch_types=[pltpu.VMEM((2, G, K), jnp.int32),
                       pltpu.VMEM((2, GK, F), jnp.float32),
                       pltpu.SemaphoreType.DMA((2,)),
                       pltpu.SemaphoreType.DMA((2,)),
                       pltpu.SemaphoreType.DMA((2,))],
        compiler_params=scp,
    )
    def _gather(nf_hbm, srcr_hbm, x_hbm, idxb, rowsb, isem, gsem, wsem):
        wid = lax.axis_index("s") * NC + lax.axis_index("c")

        def idx_copy(g, b):
            return pltpu.make_async_copy(srcr_hbm.at[pl.ds(g * G, G)],
                                         idxb.at[b], isem.at[b])

        def row_write(g, b):
            return pltpu.make_async_copy(rowsb.at[b],
                                         x_hbm.at[pl.ds(g * GK, GK)],
                                         wsem.at[b])

        def gather_drain(b):
            return pltpu.make_async_copy(nf_hbm.at[pl.ds(0, GK)],
                                         rowsb.at[b], gsem.at[b])

        @pl.when(wid < ng)
        def _():
            idx_copy(wid, 0).start()

        def body(q, carry):
            for b in (0, 1):
                gi = q * 2 + b
                g = wid + gi * NW

                @pl.when(g + NW < ng)
                def _():
                    idx_copy(g + NW, 1 - b).start()

                @pl.when((gi >= 2) & (g - 2 * NW < ng))
                def _():
                    row_write(g - 2 * NW, b).wait()

                @pl.when(g < ng)
                def _():
                    idx_copy(g, b).wait()
                    for j in range(G):
                        pltpu.async_copy(nf_hbm.at[idxb.at[b, j]],
                                         rowsb.at[b, pl.ds(j * K, K)],
                                         gsem.at[b])
                    gather_drain(b).wait()
                    row_write(g, b).start()
            return carry

        lax.fori_loop(0, pw_g // 2, body, 0)
        for t in (pw_g - 2, pw_g - 1):
            g = wid + t * NW

            @pl.when(g < ng)
            def _():
                row_write(g, t % 2).wait()

    x_src = _gather(node_feat, src_r)

    msg = _run_msg(edge_vec, x_src, W1, W2, W3)
    msg = _run_msg(edge_vec, x_src, W1, W2, W3)

    zinit = jnp.zeros((n_pad, HALF), jnp.float32)

    @functools.partial(
        pl.kernel,
        out_type=jax.ShapeDtypeStruct((NC, n_pad, HALF), jnp.float32),
        mesh=mesh,
        scratch_types=[pltpu.VMEM((2, GS, K), jnp.int32),
                       pltpu.VMEM((2, GKS, HALF), jnp.float32),
                       pltpu.VMEM_SHARED((n_pad, HALF), jnp.float32),
                       pltpu.SemaphoreType.DMA((2,)),
                       pltpu.SemaphoreType.DMA((2,)),
                       pltpu.SemaphoreType.DMA((2,))],
        compiler_params=scp,
    )
    def _scatter(dstr_hbm, msg_hbm, z_hbm, out_hbm, didxb, mb, acc_sh,
                 isem, msem, ssem):
        cid = lax.axis_index("c")
        sid = lax.axis_index("s")
        row0 = sid * rows_t
        pltpu.sync_copy(z_hbm.at[pl.ds(row0, rows_t)],
                        acc_sh.at[pl.ds(row0, rows_t)])
        plsc.subcore_barrier()

        def idx_copy(g, b):
            return pltpu.make_async_copy(dstr_hbm.at[pl.ds(g * GS, GS)],
                                         didxb.at[b], isem.at[b])

        def msg_copy(g, b):
            return pltpu.make_async_copy(
                msg_hbm.at[pl.ds(g * GKS, GKS), pl.ds(cid * HALF, HALF)],
                mb.at[b], msem.at[b])

        def scat_drain(b):
            return pltpu.make_async_copy(mb.at[b], acc_sh.at[pl.ds(0, GKS)],
                                         ssem.at[b])

        @pl.when(sid < ngs)
        def _():
            idx_copy(sid, 0).start()
            msg_copy(sid, 0).start()

        def body(q, carry):
            for b in (0, 1):
                gi = q * 2 + b
                g = sid + gi * NS

                @pl.when((gi >= 1) & (g - NS < ngs))
                def _():
                    scat_drain(1 - b).wait()

                @pl.when(g + NS < ngs)
                def _():
                    idx_copy(g + NS, 1 - b).start()
                    msg_copy(g + NS, 1 - b).start()

                @pl.when(g < ngs)
                def _():
                    idx_copy(g, b).wait()
                    msg_copy(g, b).wait()
                    for j in range(GS):
                        pltpu.async_copy(mb.at[b, pl.ds(j * K, K)],
                                         acc_sh.at[didxb.at[b, j]],
                                         ssem.at[b], add=True)
            return carry

        lax.fori_loop(0, pt_g // 2, body, 0)
        t = pt_g - 1
        g_last = sid + t * NS

        @pl.when(g_last < ngs)
        def _():
            scat_drain(t % 2).wait()
        plsc.subcore_barrier()
        pltpu.sync_copy(acc_sh.at[pl.ds(row0, rows_t)],
                        out_hbm.at[cid, pl.ds(row0, rows_t)])

    aggr2 = _scatter(dst_r, msg, zinit)

    wbig = _folded_linear(Wl0, Wl1)
    out = pl.pallas_call(
        _final_body,
        grid=(n_nodes // NB_NODE,),
        in_specs=[pl.BlockSpec((NC, NB_NODE, HALF), lambda i: (0, i, 0)),
                  pl.BlockSpec((F, F), lambda i: (0, 0)),
                  pl.BlockSpec((NB_NODE, F), lambda i: (i, 0))],
        out_specs=pl.BlockSpec((NB_NODE, F), lambda i: (i, 0)),
        out_shape=jax.ShapeDtypeStruct((n_nodes, F), jnp.float32),
    )(aggr2, wbig, node_feat)
    return out
